# baseline (speedup 1.0000x reference)
import jax
import jax.numpy as jnp
from jax import lax
from jax.experimental import pallas as pl
from jax.experimental.pallas import tpu as pltpu

B, H, W = 2, 224, 224
K, D = 19, 1024
P = H * W
KPAD = 32
DT = 64
NSTREAM = 8
DQ = DT // NSTREAM
NSTEP = B * D // DT


def _body(src_ref, colors_ref, table_ref, out_ref, onehot_ref, buf0, buf1,
          *sems):
    g = pl.program_id(0)

    @pl.when(g == 0)
    def _build_onehot():
        for b in range(B):
            q = (src_ref[b] * 127.5 + 127.5).astype(jnp.int32)
            match = None
            for c in range(3):
                eq = q[c:c + 1, :] == colors_ref[:, c:c + 1]
                match = eq if match is None else (match & eq)
            kvec = lax.broadcasted_iota(jnp.int32, (K, P), 0)
            cls = jnp.min(jnp.where(match, kvec, KPAD - 1), axis=0,
                          keepdims=True)
            onehot_ref[b] = (
                cls == lax.broadcasted_iota(jnp.int32, (KPAD, P), 0)
            ).astype(jnp.float32)

    @pl.when(g > 0)
    def _wait_prev():
        for q in range(NSTREAM):
            pltpu.make_async_copy(
                buf0.at[pl.ds(q * DQ, DQ), :],
                out_ref.at[pl.ds(q * DQ, DQ), :], sems[q]).wait()

    def _step(buf):
        b = g // (D // DT)
        d0 = g * DT - b * D
        oh = onehot_ref[b]
        tb = table_ref[pl.ds(d0, DT), :]
        buf[...] = lax.dot_general(
            tb, oh, (((1,), (0,)), ((), ())),
            preferred_element_type=jnp.float32)
        for q in range(NSTREAM):
            pltpu.make_async_copy(
                buf.at[pl.ds(q * DQ, DQ), :],
                out_ref.at[pl.ds(g * DT + q * DQ, DQ), :], sems[q]).start()

    even = lax.rem(g, 2) == 0

    @pl.when(even)
    def _even():
        _step(buf0)

    @pl.when(jnp.logical_not(even))
    def _odd():
        _step(buf1)

    @pl.when(g == NSTEP - 1)
    def _wait_last():
        for q in range(NSTREAM):
            pltpu.make_async_copy(
                buf0.at[pl.ds(q * DQ, DQ), :],
                out_ref.at[pl.ds(q * DQ, DQ), :], sems[q]).wait()


def kernel(src, colors, feats):
    src_flat = src.reshape(B, 3, P)
    colors_i = colors.astype(jnp.int32)
    table = jnp.zeros((D, KPAD), jnp.float32).at[:, :K].set(feats.T)
    out = pl.pallas_call(
        _body,
        grid=(NSTEP,),
        in_specs=[
            pl.BlockSpec((B, 3, P), lambda g: (0, 0, 0)),
            pl.BlockSpec((K, 3), lambda g: (0, 0)),
            pl.BlockSpec((D, KPAD), lambda g: (0, 0)),
        ],
        out_specs=pl.BlockSpec(memory_space=pl.ANY),
        out_shape=jax.ShapeDtypeStruct((B * D, P), jnp.float32),
        scratch_shapes=[pltpu.VMEM((B, KPAD, P), jnp.float32),
                        pltpu.VMEM((DT, P), jnp.float32),
                        pltpu.VMEM((DT, P), jnp.float32)]
                       + [pltpu.SemaphoreType.DMA] * NSTREAM,
        compiler_params=pltpu.CompilerParams(
            dimension_semantics=("arbitrary",)),
    )(src_flat, colors_i, table)
    return out.reshape(B, D, H, W)


# speedup vs baseline: 1.4007x; 1.4007x over previous
import jax
import jax.numpy as jnp
from jax import lax
from jax.experimental import pallas as pl
from jax.experimental.pallas import tpu as pltpu

B, H, W = 2, 224, 224
K, D = 19, 1024
P = H * W
KPAD = 32
DT = 32
DH = D // 2
NJ = DH // DT


def _body(src_ref, colors_ref, table_ref, out1_ref, out2_ref, onehot_ref):
    @pl.when(pl.program_id(1) == 0)
    def _build_onehot():
        q = (src_ref[0] * 127.5 + 127.5).astype(jnp.int32)
        match = None
        for c in range(3):
            eq = q[c:c + 1, :] == colors_ref[:, c:c + 1]
            match = eq if match is None else (match & eq)
        kvec = lax.broadcasted_iota(jnp.int32, (K, P), 0)
        cls = jnp.min(jnp.where(match, kvec, KPAD - 1), axis=0, keepdims=True)
        onehot_ref[...] = (
            cls == lax.broadcasted_iota(jnp.int32, (KPAD, P), 0)
        ).astype(jnp.float32)

    j = pl.program_id(1)
    oh = onehot_ref[...]
    for half, out_ref in ((0, out1_ref), (1, out2_ref)):
        tb = table_ref[pl.ds(half * DH + j * DT, DT), :]
        out_ref[0] = lax.dot_general(
            tb, oh, (((1,), (0,)), ((), ())),
            preferred_element_type=jnp.float32)


def kernel(src, colors, feats):
    src_flat = src.reshape(B, 3, P)
    colors_i = colors.astype(jnp.int32)
    table = jnp.zeros((D, KPAD), jnp.float32).at[:, :K].set(feats.T)
    o1, o2 = pl.pallas_call(
        _body,
        grid=(B, NJ),
        in_specs=[
            pl.BlockSpec((1, 3, P), lambda b, j: (b, 0, 0)),
            pl.BlockSpec((K, 3), lambda b, j: (0, 0)),
            pl.BlockSpec((D, KPAD), lambda b, j: (0, 0)),
        ],
        out_specs=[pl.BlockSpec((1, DT, P), lambda b, j: (b, j, 0)),
                   pl.BlockSpec((1, DT, P), lambda b, j: (b, j, 0))],
        out_shape=[jax.ShapeDtypeStruct((B, DH, P), jnp.float32),
                   jax.ShapeDtypeStruct((B, DH, P), jnp.float32)],
        scratch_shapes=[pltpu.VMEM((KPAD, P), jnp.float32)],
        compiler_params=pltpu.CompilerParams(
            dimension_semantics=("arbitrary", "arbitrary")),
    )(src_flat, colors_i, table)
    out = jnp.concatenate([o1, o2], axis=1)
    return out.reshape(B, D, H, W)
